# Initial kernel scaffold; baseline (speedup 1.0000x reference)
import jax
import jax.numpy as jnp
from jax.experimental import pallas as pl


def kernel(x, edge_index, Wl, Wr, b_sage, Wf1, bf1, Wout, bout):
    raise NotImplementedError("write your pallas kernel here")


# speedup vs baseline: 39.3033x; 39.3033x over previous
import functools

import jax
import jax.numpy as jnp
from jax import lax
from jax.experimental import pallas as pl
from jax.experimental.pallas import tpu as pltpu
from jax.experimental.pallas import tpu_sc as plsc

NC = 2
NS = 16
NW = NC * NS
L = 16


def _tc_scalars_body(x_ref, wl_ref, wr_ref, wf1_ref, wout_ref, bs_ref,
                     bf1_ref, bo_ref, gt_ref):
    hp = lax.Precision.HIGHEST
    dot = functools.partial(lax.dot_general, precision=hp,
                            preferred_element_type=jnp.float32)
    a = dot(wf1_ref[...], wout_ref[...], (((1,), (0,)), ((), ())))
    a1 = a[:128, :]
    a2 = a[128:, :]
    p1 = dot(wl_ref[...], a1, (((1,), (0,)), ((), ())))
    p2 = dot(wl_ref[...], a2, (((1,), (0,)), ((), ())))
    q1 = dot(wr_ref[...], a1, (((1,), (0,)), ((), ())))
    q2 = dot(wr_ref[...], a2, (((1,), (0,)), ((), ())))
    pmat = jnp.concatenate(
        [p1, p2, q1, q2, jnp.zeros((128, 4), jnp.float32)], axis=1)
    gt = dot(pmat, x_ref[...], (((0,), (1,)), ((), ())))
    cu = (dot(bs_ref[...], a1, (((1,), (0,)), ((), ())))[0, 0]
          + dot(bf1_ref[...], wout_ref[...], (((1,), (0,)), ((), ())))[0, 0]
          + bo_ref[0, 0])
    cv = dot(bs_ref[...], a2, (((1,), (0,)), ((), ())))[0, 0]
    row = lax.broadcasted_iota(jnp.int32, (8, 1), 0)
    bias = jnp.where(row == 2, cu, 0.0) + jnp.where(row == 3, cv, 0.0)
    gt_ref[...] = gt + bias


def _make_tc_scalars(n_pad):
    return pl.pallas_call(
        _tc_scalars_body,
        out_shape=jax.ShapeDtypeStruct((8, n_pad), jnp.float32),
    )


def _make_scatter(e_per_w, n_pad):
    mesh = plsc.VectorSubcoreMesh(core_axis_name="c", subcore_axis_name="s")

    @functools.partial(
        pl.kernel, mesh=mesh,
        compiler_params=pltpu.CompilerParams(needs_layout_passes=False, use_tc_tiling_on_sc=False),
        out_type=[jax.ShapeDtypeStruct((NW, n_pad), jnp.float32)] * 3,
        scratch_types=[
            pltpu.VMEM((e_per_w,), jnp.int32),
            pltpu.VMEM((e_per_w,), jnp.int32),
            pltpu.VMEM((n_pad,), jnp.float32),
            pltpu.VMEM((n_pad,), jnp.float32),
            pltpu.VMEM((n_pad,), jnp.float32),
            pltpu.VMEM((n_pad,), jnp.float32),
            pltpu.VMEM((n_pad,), jnp.float32),
        ],
    )
    def scatter_kernel(src_hbm, dst_hbm, gt_hbm, zeros_hbm,
                       s1_out, s2_out, cnt_out,
                       src_v, dst_v, gl1_v, gl2_v, s1_v, s2_v, cnt_v):
        wid = lax.axis_index("s") * NC + lax.axis_index("c")
        base = wid * e_per_w
        pltpu.sync_copy(src_hbm.at[pl.ds(base, e_per_w)], src_v)
        pltpu.sync_copy(dst_hbm.at[pl.ds(base, e_per_w)], dst_v)
        pltpu.sync_copy(gt_hbm.at[0], gl1_v)
        pltpu.sync_copy(gt_hbm.at[1], gl2_v)
        pltpu.sync_copy(zeros_hbm, s1_v)
        pltpu.sync_copy(zeros_hbm, s2_v)
        pltpu.sync_copy(zeros_hbm, cnt_v)
        ones = jnp.full((L,), 1.0, jnp.float32)

        def body(i, carry):
            sv = src_v[pl.ds(i * L, L)]
            dv = dst_v[pl.ds(i * L, L)]
            g1 = plsc.load_gather(gl1_v, [sv])
            g2 = plsc.load_gather(gl2_v, [sv])
            plsc.addupdate_scatter(s1_v, [dv], g1)
            plsc.addupdate_scatter(s2_v, [dv], g2)
            plsc.addupdate_scatter(cnt_v, [dv], ones)
            return carry

        lax.fori_loop(0, e_per_w // L, body, 0)
        pltpu.sync_copy(s1_v, s1_out.at[wid])
        pltpu.sync_copy(s2_v, s2_out.at[wid])
        pltpu.sync_copy(cnt_v, cnt_out.at[wid])

    return scatter_kernel


def _make_reduce(n_pad):
    n_per_w = n_pad // NW
    mesh = plsc.VectorSubcoreMesh(core_axis_name="c", subcore_axis_name="s")

    @functools.partial(
        pl.kernel, mesh=mesh,
        compiler_params=pltpu.CompilerParams(needs_layout_passes=False, use_tc_tiling_on_sc=False),
        out_type=jax.ShapeDtypeStruct((2, n_pad), jnp.float32),
        scratch_types=[
            pltpu.VMEM((NW, n_per_w), jnp.float32),
            pltpu.VMEM((n_per_w,), jnp.float32),
            pltpu.VMEM((n_per_w,), jnp.float32),
            pltpu.VMEM((n_per_w,), jnp.float32),
            pltpu.VMEM((n_per_w,), jnp.float32),
        ],
    )
    def reduce_kernel(s1_hbm, s2_hbm, cnt_hbm, gt_hbm, uv_out,
                      part_v, u_v, v_v, cnt_v, g_v):
        wid = lax.axis_index("s") * NC + lax.axis_index("c")
        col = wid * n_per_w

        def sum_partials(hbm, acc_v):
            pltpu.sync_copy(hbm.at[:, pl.ds(col, n_per_w)], part_v)

            def body(i, carry):
                acc = jnp.zeros((L,), jnp.float32)
                for t in range(NW):
                    acc = acc + part_v[t, pl.ds(i * L, L)]
                acc_v[pl.ds(i * L, L)] = acc
                return carry

            lax.fori_loop(0, n_per_w // L, body, 0)

        sum_partials(cnt_hbm, cnt_v)
        sum_partials(s1_hbm, u_v)
        sum_partials(s2_hbm, v_v)

        def finish(gt_row, acc_v):
            pltpu.sync_copy(gt_hbm.at[gt_row, pl.ds(col, n_per_w)], g_v)

            def body(i, carry):
                sl = pl.ds(i * L, L)
                den = jnp.maximum(cnt_v[sl], 1.0)
                acc_v[sl] = acc_v[sl] / den + g_v[sl]
                return carry

            lax.fori_loop(0, n_per_w // L, body, 0)

        finish(2, u_v)
        finish(3, v_v)
        pltpu.sync_copy(u_v, uv_out.at[0, pl.ds(col, n_per_w)])
        pltpu.sync_copy(v_v, uv_out.at[1, pl.ds(col, n_per_w)])

    return reduce_kernel


def _make_gather(e_per_w, n_pad, n_edges):
    mesh = plsc.VectorSubcoreMesh(core_axis_name="c", subcore_axis_name="s")

    @functools.partial(
        pl.kernel, mesh=mesh,
        compiler_params=pltpu.CompilerParams(needs_layout_passes=False, use_tc_tiling_on_sc=False),
        out_type=jax.ShapeDtypeStruct((n_edges,), jnp.float32),
        scratch_types=[
            pltpu.VMEM((e_per_w,), jnp.int32),
            pltpu.VMEM((e_per_w,), jnp.int32),
            pltpu.VMEM((n_pad,), jnp.float32),
            pltpu.VMEM((n_pad,), jnp.float32),
            pltpu.VMEM((e_per_w,), jnp.float32),
        ],
    )
    def gather_kernel(src_hbm, dst_hbm, uv_hbm, out_hbm,
                      src_v, dst_v, u_v, v_v, o_v):
        wid = lax.axis_index("s") * NC + lax.axis_index("c")
        base = wid * e_per_w
        pltpu.sync_copy(src_hbm.at[pl.ds(base, e_per_w)], src_v)
        pltpu.sync_copy(dst_hbm.at[pl.ds(base, e_per_w)], dst_v)
        pltpu.sync_copy(uv_hbm.at[0], u_v)
        pltpu.sync_copy(uv_hbm.at[1], v_v)

        def body(i, carry):
            sl = pl.ds(i * L, L)
            gu = plsc.load_gather(u_v, [src_v[sl]])
            gv = plsc.load_gather(v_v, [dst_v[sl]])
            o_v[sl] = gu + gv
            return carry

        lax.fori_loop(0, e_per_w // L, body, 0)
        pltpu.sync_copy(o_v, out_hbm.at[pl.ds(base, e_per_w)])

    return gather_kernel


def kernel(x, edge_index, Wl, Wr, b_sage, Wf1, bf1, Wout, bout):
    n, d = x.shape
    n_edges = edge_index.shape[1]
    n_pad = ((n + NW * L - 1) // (NW * L)) * (NW * L)
    e_per_w = n_edges // NW
    assert e_per_w * NW == n_edges and e_per_w % L == 0

    src = edge_index[0]
    dst = edge_index[1]
    x_pad = jnp.zeros((n_pad, d), jnp.float32).at[:n].set(x)

    gt = _make_tc_scalars(n_pad)(
        x_pad, Wl, Wr, Wf1, Wout,
        b_sage.reshape(1, -1), bf1.reshape(1, -1), bout.reshape(1, 1))

    zeros_n = jnp.zeros((n_pad,), jnp.float32)
    s1, s2, cnt = _make_scatter(e_per_w, n_pad)(src, dst, gt, zeros_n)
    uv = _make_reduce(n_pad)(s1, s2, cnt, gt)
    out = _make_gather(e_per_w, n_pad, n_edges)(src, dst, uv)
    return out.reshape(n_edges, 1)


# speedup vs baseline: 41.8911x; 1.0658x over previous
import functools

import jax
import jax.numpy as jnp
from jax import lax
from jax.experimental import pallas as pl
from jax.experimental.pallas import tpu as pltpu
from jax.experimental.pallas import tpu_sc as plsc

NC = 2
NS = 16
NW = NC * NS
L = 16


def _tc_scalars_body(x_ref, wl_ref, wr_ref, wf1_ref, wout_ref, bs_ref,
                     bf1_ref, bo_ref, gt_ref):
    hp = lax.Precision.HIGHEST
    dot = functools.partial(lax.dot_general, precision=hp,
                            preferred_element_type=jnp.float32)
    a = dot(wf1_ref[...], wout_ref[...], (((1,), (0,)), ((), ())))
    a1 = a[:128, :]
    a2 = a[128:, :]
    p1 = dot(wl_ref[...], a1, (((1,), (0,)), ((), ())))
    p2 = dot(wl_ref[...], a2, (((1,), (0,)), ((), ())))
    q1 = dot(wr_ref[...], a1, (((1,), (0,)), ((), ())))
    q2 = dot(wr_ref[...], a2, (((1,), (0,)), ((), ())))
    pmat = jnp.concatenate(
        [p1, p2, q1, q2, jnp.zeros((128, 4), jnp.float32)], axis=1)
    gt = dot(pmat, x_ref[...], (((0,), (1,)), ((), ())))
    cu = (dot(bs_ref[...], a1, (((1,), (0,)), ((), ())))[0, 0]
          + dot(bf1_ref[...], wout_ref[...], (((1,), (0,)), ((), ())))[0, 0]
          + bo_ref[0, 0])
    cv = dot(bs_ref[...], a2, (((1,), (0,)), ((), ())))[0, 0]
    row = lax.broadcasted_iota(jnp.int32, (8, 1), 0)
    bias = jnp.where(row == 2, cu, 0.0) + jnp.where(row == 3, cv, 0.0)
    gt_ref[...] = gt + bias


def _make_tc_scalars(n_pad):
    return pl.pallas_call(
        _tc_scalars_body,
        out_shape=jax.ShapeDtypeStruct((8, n_pad), jnp.float32),
    )


def _make_scatter(e_per_w, n_pad):
    mesh = plsc.VectorSubcoreMesh(core_axis_name="c", subcore_axis_name="s")

    @functools.partial(
        pl.kernel, mesh=mesh,
        compiler_params=pltpu.CompilerParams(needs_layout_passes=False, use_tc_tiling_on_sc=False),
        out_type=[jax.ShapeDtypeStruct((NW, n_pad), jnp.float32)] * 3,
        scratch_types=[
            pltpu.VMEM((e_per_w,), jnp.int32),
            pltpu.VMEM((e_per_w,), jnp.int32),
            pltpu.VMEM((n_pad,), jnp.float32),
            pltpu.VMEM((n_pad,), jnp.float32),
            pltpu.VMEM((n_pad,), jnp.float32),
            pltpu.VMEM((n_pad,), jnp.float32),
            pltpu.VMEM((n_pad,), jnp.float32),
        ],
    )
    def scatter_kernel(src_hbm, dst_hbm, gt_hbm, zeros_hbm,
                       s1_out, s2_out, cnt_out,
                       src_v, dst_v, gl1_v, gl2_v, s1_v, s2_v, cnt_v):
        wid = lax.axis_index("s") * NC + lax.axis_index("c")
        base = wid * e_per_w
        pltpu.sync_copy(src_hbm.at[pl.ds(base, e_per_w)], src_v)
        pltpu.sync_copy(dst_hbm.at[pl.ds(base, e_per_w)], dst_v)
        pltpu.sync_copy(gt_hbm.at[0], gl1_v)
        pltpu.sync_copy(gt_hbm.at[1], gl2_v)
        pltpu.sync_copy(zeros_hbm, s1_v)
        pltpu.sync_copy(zeros_hbm, s2_v)
        pltpu.sync_copy(zeros_hbm, cnt_v)
        ones = jnp.full((L,), 1.0, jnp.float32)

        @plsc.parallel_loop(0, e_per_w // L, unroll=8)
        def _(i):
            sv = src_v[pl.ds(i * L, L)]
            dv = dst_v[pl.ds(i * L, L)]
            g1 = plsc.load_gather(gl1_v, [sv])
            g2 = plsc.load_gather(gl2_v, [sv])
            plsc.addupdate_scatter(s1_v, [dv], g1)
            plsc.addupdate_scatter(s2_v, [dv], g2)
            plsc.addupdate_scatter(cnt_v, [dv], ones)
        pltpu.sync_copy(s1_v, s1_out.at[wid])
        pltpu.sync_copy(s2_v, s2_out.at[wid])
        pltpu.sync_copy(cnt_v, cnt_out.at[wid])

    return scatter_kernel


def _make_reduce(n_pad):
    n_per_w = n_pad // NW
    mesh = plsc.VectorSubcoreMesh(core_axis_name="c", subcore_axis_name="s")

    @functools.partial(
        pl.kernel, mesh=mesh,
        compiler_params=pltpu.CompilerParams(needs_layout_passes=False, use_tc_tiling_on_sc=False),
        out_type=jax.ShapeDtypeStruct((2, n_pad), jnp.float32),
        scratch_types=[
            pltpu.VMEM((NW, n_per_w), jnp.float32),
            pltpu.VMEM((n_per_w,), jnp.float32),
            pltpu.VMEM((n_per_w,), jnp.float32),
            pltpu.VMEM((n_per_w,), jnp.float32),
            pltpu.VMEM((n_per_w,), jnp.float32),
        ],
    )
    def reduce_kernel(s1_hbm, s2_hbm, cnt_hbm, gt_hbm, uv_out,
                      part_v, u_v, v_v, cnt_v, g_v):
        wid = lax.axis_index("s") * NC + lax.axis_index("c")
        col = wid * n_per_w

        def sum_partials(hbm, acc_v):
            pltpu.sync_copy(hbm.at[:, pl.ds(col, n_per_w)], part_v)

            def body(i, carry):
                acc = jnp.zeros((L,), jnp.float32)
                for t in range(NW):
                    acc = acc + part_v[t, pl.ds(i * L, L)]
                acc_v[pl.ds(i * L, L)] = acc
                return carry

            lax.fori_loop(0, n_per_w // L, body, 0)

        sum_partials(cnt_hbm, cnt_v)
        sum_partials(s1_hbm, u_v)
        sum_partials(s2_hbm, v_v)

        def finish(gt_row, acc_v):
            pltpu.sync_copy(gt_hbm.at[gt_row, pl.ds(col, n_per_w)], g_v)

            def body(i, carry):
                sl = pl.ds(i * L, L)
                den = jnp.maximum(cnt_v[sl], 1.0)
                acc_v[sl] = acc_v[sl] / den + g_v[sl]
                return carry

            lax.fori_loop(0, n_per_w // L, body, 0)

        finish(2, u_v)
        finish(3, v_v)
        pltpu.sync_copy(u_v, uv_out.at[0, pl.ds(col, n_per_w)])
        pltpu.sync_copy(v_v, uv_out.at[1, pl.ds(col, n_per_w)])

    return reduce_kernel


def _make_gather(e_per_w, n_pad, n_edges):
    mesh = plsc.VectorSubcoreMesh(core_axis_name="c", subcore_axis_name="s")

    @functools.partial(
        pl.kernel, mesh=mesh,
        compiler_params=pltpu.CompilerParams(needs_layout_passes=False, use_tc_tiling_on_sc=False),
        out_type=jax.ShapeDtypeStruct((n_edges,), jnp.float32),
        scratch_types=[
            pltpu.VMEM((e_per_w,), jnp.int32),
            pltpu.VMEM((e_per_w,), jnp.int32),
            pltpu.VMEM((n_pad,), jnp.float32),
            pltpu.VMEM((n_pad,), jnp.float32),
            pltpu.VMEM((e_per_w,), jnp.float32),
        ],
    )
    def gather_kernel(src_hbm, dst_hbm, uv_hbm, out_hbm,
                      src_v, dst_v, u_v, v_v, o_v):
        wid = lax.axis_index("s") * NC + lax.axis_index("c")
        base = wid * e_per_w
        pltpu.sync_copy(src_hbm.at[pl.ds(base, e_per_w)], src_v)
        pltpu.sync_copy(dst_hbm.at[pl.ds(base, e_per_w)], dst_v)
        pltpu.sync_copy(uv_hbm.at[0], u_v)
        pltpu.sync_copy(uv_hbm.at[1], v_v)

        @plsc.parallel_loop(0, e_per_w // L, unroll=8)
        def _(i):
            sl = pl.ds(i * L, L)
            gu = plsc.load_gather(u_v, [src_v[sl]])
            gv = plsc.load_gather(v_v, [dst_v[sl]])
            o_v[sl] = gu + gv
        pltpu.sync_copy(o_v, out_hbm.at[pl.ds(base, e_per_w)])

    return gather_kernel


def kernel(x, edge_index, Wl, Wr, b_sage, Wf1, bf1, Wout, bout):
    n, d = x.shape
    n_edges = edge_index.shape[1]
    n_pad = ((n + NW * L - 1) // (NW * L)) * (NW * L)
    e_per_w = n_edges // NW
    assert e_per_w * NW == n_edges and e_per_w % L == 0

    src = edge_index[0]
    dst = edge_index[1]
    x_pad = jnp.zeros((n_pad, d), jnp.float32).at[:n].set(x)

    gt = _make_tc_scalars(n_pad)(
        x_pad, Wl, Wr, Wf1, Wout,
        b_sage.reshape(1, -1), bf1.reshape(1, -1), bout.reshape(1, 1))

    zeros_n = jnp.zeros((n_pad,), jnp.float32)
    s1, s2, cnt = _make_scatter(e_per_w, n_pad)(src, dst, gt, zeros_n)
    uv = _make_reduce(n_pad)(s1, s2, cnt, gt)
    out = _make_gather(e_per_w, n_pad, n_edges)(src, dst, uv)
    return out.reshape(n_edges, 1)


# speedup vs baseline: 45.9083x; 1.0959x over previous
import functools

import jax
import jax.numpy as jnp
from jax import lax
from jax.experimental import pallas as pl
from jax.experimental.pallas import tpu as pltpu
from jax.experimental.pallas import tpu_sc as plsc

NC = 2
NS = 16
NW = NC * NS
L = 16

_SC_PARAMS = pltpu.CompilerParams(
    needs_layout_passes=False, use_tc_tiling_on_sc=False)


def _tc_scalars_body(x_ref, wl_ref, wr_ref, wf1_ref, wout_ref, bs_ref,
                     bf1_ref, bo_ref, gt_ref):
    hp = lax.Precision.HIGHEST
    dot = functools.partial(lax.dot_general, precision=hp,
                            preferred_element_type=jnp.float32)
    a = dot(wf1_ref[...], wout_ref[...], (((1,), (0,)), ((), ())))
    a1 = a[:128, :]
    a2 = a[128:, :]
    p1 = dot(wl_ref[...], a1, (((1,), (0,)), ((), ())))
    p2 = dot(wl_ref[...], a2, (((1,), (0,)), ((), ())))
    q1 = dot(wr_ref[...], a1, (((1,), (0,)), ((), ())))
    q2 = dot(wr_ref[...], a2, (((1,), (0,)), ((), ())))
    pmat = jnp.concatenate(
        [p1, p2, q1, q2, jnp.zeros((128, 4), jnp.float32)], axis=1)
    gt = dot(pmat, x_ref[...], (((0,), (1,)), ((), ())))
    cu = (dot(bs_ref[...], a1, (((1,), (0,)), ((), ())))[0, 0]
          + dot(bf1_ref[...], wout_ref[...], (((1,), (0,)), ((), ())))[0, 0]
          + bo_ref[0, 0])
    cv = dot(bs_ref[...], a2, (((1,), (0,)), ((), ())))[0, 0]
    row = lax.broadcasted_iota(jnp.int32, (8, 1), 0)
    bias = jnp.where(row == 2, cu, 0.0) + jnp.where(row == 3, cv, 0.0)
    gt_ref[...] = gt + bias


def _make_tc_scalars(n_pad):
    return pl.pallas_call(
        _tc_scalars_body,
        out_shape=jax.ShapeDtypeStruct((8, n_pad), jnp.float32),
    )


def _make_scatter(e_per_w, n_pad):
    n_sl = n_pad // NS
    mesh = plsc.VectorSubcoreMesh(core_axis_name="c", subcore_axis_name="s")

    @functools.partial(
        pl.kernel, mesh=mesh,
        compiler_params=_SC_PARAMS,
        out_type=jax.ShapeDtypeStruct((2 * 3, n_pad), jnp.float32),
        scratch_types=[
            pltpu.VMEM((e_per_w,), jnp.int32),
            pltpu.VMEM((e_per_w,), jnp.int32),
            pltpu.VMEM((n_pad,), jnp.float32),
            pltpu.VMEM((n_pad,), jnp.float32),
            pltpu.VMEM((n_pad,), jnp.float32),
            pltpu.VMEM((n_pad,), jnp.float32),
            pltpu.VMEM((n_pad,), jnp.float32),
            pltpu.VMEM((NS, n_sl), jnp.float32),
            pltpu.VMEM((n_sl,), jnp.float32),
            pltpu.MemorySpace.VMEM_SHARED((3, NS, n_pad), jnp.float32),
        ],
    )
    def scatter_kernel(src_hbm, dst_hbm, gt_hbm, zeros_hbm, part_out,
                       src_v, dst_v, gl1_v, gl2_v, s1_v, s2_v, cnt_v,
                       stage_v, red_v, sh):
        cid = lax.axis_index("c")
        tid = lax.axis_index("s")
        wid = tid * NC + cid
        base = wid * e_per_w
        pltpu.sync_copy(src_hbm.at[pl.ds(base, e_per_w)], src_v)
        pltpu.sync_copy(dst_hbm.at[pl.ds(base, e_per_w)], dst_v)
        pltpu.sync_copy(gt_hbm.at[0], gl1_v)
        pltpu.sync_copy(gt_hbm.at[1], gl2_v)
        pltpu.sync_copy(zeros_hbm, s1_v)
        pltpu.sync_copy(zeros_hbm, s2_v)
        pltpu.sync_copy(zeros_hbm, cnt_v)
        ones = jnp.full((L,), 1.0, jnp.float32)

        @plsc.parallel_loop(0, e_per_w // L, unroll=8)
        def _(i):
            sv = src_v[pl.ds(i * L, L)]
            dv = dst_v[pl.ds(i * L, L)]
            g1 = plsc.load_gather(gl1_v, [sv])
            g2 = plsc.load_gather(gl2_v, [sv])
            plsc.addupdate_scatter(s1_v, [dv], g1)
            plsc.addupdate_scatter(s2_v, [dv], g2)
            plsc.addupdate_scatter(cnt_v, [dv], ones)

        pltpu.sync_copy(s1_v, sh.at[0, tid])
        pltpu.sync_copy(s2_v, sh.at[1, tid])
        pltpu.sync_copy(cnt_v, sh.at[2, tid])
        plsc.subcore_barrier()

        col = tid * n_sl
        for arr in range(3):
            pltpu.sync_copy(sh.at[arr, :, pl.ds(col, n_sl)], stage_v)

            @plsc.parallel_loop(0, n_sl // L, unroll=2)
            def _(i):
                sl = pl.ds(i * L, L)
                acc = stage_v[0, sl]
                for t in range(1, NS):
                    acc = acc + stage_v[t, sl]
                red_v[sl] = acc

            pltpu.sync_copy(red_v, part_out.at[cid * 3 + arr, pl.ds(col, n_sl)])

    return scatter_kernel


def _make_uv_gather(e_per_w, n_pad, n_edges):
    n_sl = n_pad // NS
    mesh = plsc.VectorSubcoreMesh(core_axis_name="c", subcore_axis_name="s")

    @functools.partial(
        pl.kernel, mesh=mesh,
        compiler_params=_SC_PARAMS,
        out_type=jax.ShapeDtypeStruct((n_edges,), jnp.float32),
        scratch_types=[
            pltpu.VMEM((6, n_sl), jnp.float32),
            pltpu.VMEM((2, n_sl), jnp.float32),
            pltpu.VMEM((n_sl,), jnp.float32),
            pltpu.VMEM((n_sl,), jnp.float32),
            pltpu.VMEM((e_per_w,), jnp.int32),
            pltpu.VMEM((e_per_w,), jnp.int32),
            pltpu.VMEM((n_pad,), jnp.float32),
            pltpu.VMEM((n_pad,), jnp.float32),
            pltpu.VMEM((e_per_w,), jnp.float32),
            pltpu.MemorySpace.VMEM_SHARED((2, n_pad), jnp.float32),
        ],
    )
    def uv_gather_kernel(part_hbm, gt_hbm, src_hbm, dst_hbm, out_hbm,
                         part_v, g_v, u_t, v_t, src_v, dst_v, u_v, v_v, o_v,
                         sh_uv):
        cid = lax.axis_index("c")
        tid = lax.axis_index("s")
        wid = tid * NC + cid
        col = tid * n_sl
        pltpu.sync_copy(part_hbm.at[:, pl.ds(col, n_sl)], part_v)
        pltpu.sync_copy(gt_hbm.at[pl.ds(2, 2), pl.ds(col, n_sl)], g_v)

        @plsc.parallel_loop(0, n_sl // L, unroll=2)
        def _(i):
            sl = pl.ds(i * L, L)
            den = jnp.maximum(part_v[2, sl] + part_v[5, sl], 1.0)
            u_t[sl] = (part_v[0, sl] + part_v[3, sl]) / den + g_v[0, sl]
            v_t[sl] = (part_v[1, sl] + part_v[4, sl]) / den + g_v[1, sl]

        pltpu.sync_copy(u_t, sh_uv.at[0, pl.ds(col, n_sl)])
        pltpu.sync_copy(v_t, sh_uv.at[1, pl.ds(col, n_sl)])
        plsc.subcore_barrier()

        pltpu.sync_copy(sh_uv.at[0], u_v)
        pltpu.sync_copy(sh_uv.at[1], v_v)
        base = wid * e_per_w
        pltpu.sync_copy(src_hbm.at[pl.ds(base, e_per_w)], src_v)
        pltpu.sync_copy(dst_hbm.at[pl.ds(base, e_per_w)], dst_v)

        @plsc.parallel_loop(0, e_per_w // L, unroll=8)
        def _(i):
            sl = pl.ds(i * L, L)
            gu = plsc.load_gather(u_v, [src_v[sl]])
            gv = plsc.load_gather(v_v, [dst_v[sl]])
            o_v[sl] = gu + gv

        pltpu.sync_copy(o_v, out_hbm.at[pl.ds(base, e_per_w)])

    return uv_gather_kernel


def kernel(x, edge_index, Wl, Wr, b_sage, Wf1, bf1, Wout, bout):
    n, d = x.shape
    n_edges = edge_index.shape[1]
    n_pad = ((n + NW * L - 1) // (NW * L)) * (NW * L)
    e_per_w = n_edges // NW
    assert e_per_w * NW == n_edges and e_per_w % L == 0

    src = edge_index[0]
    dst = edge_index[1]
    x_pad = jnp.zeros((n_pad, d), jnp.float32).at[:n].set(x)

    gt = _make_tc_scalars(n_pad)(
        x_pad, Wl, Wr, Wf1, Wout,
        b_sage.reshape(1, -1), bf1.reshape(1, -1), bout.reshape(1, 1))

    zeros_n = jnp.zeros((n_pad,), jnp.float32)
    part = _make_scatter(e_per_w, n_pad)(src, dst, gt, zeros_n)
    out = _make_uv_gather(e_per_w, n_pad, n_edges)(part, gt, src, dst)
    return out.reshape(n_edges, 1)


# speedup vs baseline: 59.1030x; 1.2874x over previous
import functools

import jax
import jax.numpy as jnp
from jax import lax
from jax.experimental import pallas as pl
from jax.experimental.pallas import tpu as pltpu
from jax.experimental.pallas import tpu_sc as plsc

NC = 2
NS = 16
NW = NC * NS
L = 16

_SC_PARAMS = pltpu.CompilerParams(
    needs_layout_passes=False, use_tc_tiling_on_sc=False)


def _tc_scalars_body(n_pad, x_ref, wl_ref, wr_ref, wf1_ref, wout_ref, bs_ref,
                     bf1_ref, bo_ref, gt_ref):
    hp = lax.Precision.HIGHEST
    dot = functools.partial(lax.dot_general, precision=hp,
                            preferred_element_type=jnp.float32)
    a = dot(wf1_ref[...], wout_ref[...], (((1,), (0,)), ((), ())))
    a1 = a[:128, :]
    a2 = a[128:, :]
    p1 = dot(wl_ref[...], a1, (((1,), (0,)), ((), ())))
    p2 = dot(wl_ref[...], a2, (((1,), (0,)), ((), ())))
    q1 = dot(wr_ref[...], a1, (((1,), (0,)), ((), ())))
    q2 = dot(wr_ref[...], a2, (((1,), (0,)), ((), ())))
    pmat = jnp.concatenate(
        [p1, p2, q1, q2, jnp.zeros((128, 4), jnp.float32)], axis=1)
    x = x_ref[...]
    n = x.shape[0]
    if n < n_pad:
        x = jnp.concatenate(
            [x, jnp.zeros((n_pad - n, x.shape[1]), jnp.float32)], axis=0)
    gt = dot(pmat, x, (((0,), (1,)), ((), ())))
    cu = (dot(bs_ref[...], a1, (((1,), (0,)), ((), ())))[0, 0]
          + dot(bf1_ref[...], wout_ref[...], (((1,), (0,)), ((), ())))[0, 0]
          + bo_ref[0, 0])
    cv = dot(bs_ref[...], a2, (((1,), (0,)), ((), ())))[0, 0]
    row = lax.broadcasted_iota(jnp.int32, (8, 1), 0)
    bias = jnp.where(row == 2, cu, 0.0) + jnp.where(row == 3, cv, 0.0)
    gt_ref[...] = gt + bias


def _make_tc_scalars(n_pad):
    return pl.pallas_call(
        functools.partial(_tc_scalars_body, n_pad),
        out_shape=jax.ShapeDtypeStruct((8, n_pad), jnp.float32),
    )


def _make_scatter(e_per_w, n_pad):
    n_sl = n_pad // NS
    mesh = plsc.VectorSubcoreMesh(core_axis_name="c", subcore_axis_name="s")

    @functools.partial(
        pl.kernel, mesh=mesh,
        compiler_params=_SC_PARAMS,
        out_type=jax.ShapeDtypeStruct((2 * 3, n_pad), jnp.float32),
        scratch_types=[
            pltpu.VMEM((1, e_per_w), jnp.int32),
            pltpu.VMEM((1, e_per_w), jnp.int32),
            pltpu.VMEM((n_pad,), jnp.float32),
            pltpu.VMEM((n_pad,), jnp.float32),
            pltpu.VMEM((n_pad,), jnp.float32),
            pltpu.VMEM((n_pad,), jnp.float32),
            pltpu.VMEM((n_pad,), jnp.float32),
            pltpu.VMEM((NS, n_sl), jnp.float32),
            pltpu.VMEM((n_sl,), jnp.float32),
            pltpu.MemorySpace.VMEM_SHARED((3, NS, n_pad), jnp.float32),
        ],
    )
    def scatter_kernel(ei_hbm, gt_hbm, part_out,
                       src_v, dst_v, gl1_v, gl2_v, s1_v, s2_v, cnt_v,
                       stage_v, red_v, sh):
        cid = lax.axis_index("c")
        tid = lax.axis_index("s")
        wid = tid * NC + cid
        base = wid * e_per_w
        pltpu.sync_copy(ei_hbm.at[pl.ds(0, 1), pl.ds(base, e_per_w)], src_v)
        pltpu.sync_copy(ei_hbm.at[pl.ds(1, 1), pl.ds(base, e_per_w)], dst_v)
        pltpu.sync_copy(gt_hbm.at[0], gl1_v)
        pltpu.sync_copy(gt_hbm.at[1], gl2_v)
        zeros = jnp.zeros((L,), jnp.float32)

        @plsc.parallel_loop(0, n_pad // L, unroll=8)
        def _(i):
            sl = pl.ds(i * L, L)
            s1_v[sl] = zeros
            s2_v[sl] = zeros
            cnt_v[sl] = zeros

        ones = jnp.full((L,), 1.0, jnp.float32)

        @plsc.parallel_loop(0, e_per_w // L, unroll=8)
        def _(i):
            sv = src_v[0, pl.ds(i * L, L)]
            dv = dst_v[0, pl.ds(i * L, L)]
            g1 = plsc.load_gather(gl1_v, [sv])
            g2 = plsc.load_gather(gl2_v, [sv])
            plsc.addupdate_scatter(s1_v, [dv], g1)
            plsc.addupdate_scatter(s2_v, [dv], g2)
            plsc.addupdate_scatter(cnt_v, [dv], ones)

        pltpu.sync_copy(s1_v, sh.at[0, tid])
        pltpu.sync_copy(s2_v, sh.at[1, tid])
        pltpu.sync_copy(cnt_v, sh.at[2, tid])
        plsc.subcore_barrier()

        col = tid * n_sl
        for arr in range(3):
            pltpu.sync_copy(sh.at[arr, :, pl.ds(col, n_sl)], stage_v)

            @plsc.parallel_loop(0, n_sl // L, unroll=2)
            def _(i):
                sl = pl.ds(i * L, L)
                acc = stage_v[0, sl]
                for t in range(1, NS):
                    acc = acc + stage_v[t, sl]
                red_v[sl] = acc

            pltpu.sync_copy(red_v, part_out.at[cid * 3 + arr, pl.ds(col, n_sl)])

    return scatter_kernel


def _make_uv_gather(e_per_w, n_pad, n_edges):
    n_sl = n_pad // NS
    mesh = plsc.VectorSubcoreMesh(core_axis_name="c", subcore_axis_name="s")

    @functools.partial(
        pl.kernel, mesh=mesh,
        compiler_params=_SC_PARAMS,
        out_type=jax.ShapeDtypeStruct((n_edges,), jnp.float32),
        scratch_types=[
            pltpu.VMEM((6, n_sl), jnp.float32),
            pltpu.VMEM((2, n_sl), jnp.float32),
            pltpu.VMEM((n_sl,), jnp.float32),
            pltpu.VMEM((n_sl,), jnp.float32),
            pltpu.VMEM((1, e_per_w), jnp.int32),
            pltpu.VMEM((1, e_per_w), jnp.int32),
            pltpu.VMEM((n_pad,), jnp.float32),
            pltpu.VMEM((n_pad,), jnp.float32),
            pltpu.VMEM((e_per_w,), jnp.float32),
            pltpu.MemorySpace.VMEM_SHARED((2, n_pad), jnp.float32),
        ],
    )
    def uv_gather_kernel(part_hbm, gt_hbm, ei_hbm, out_hbm,
                         part_v, g_v, u_t, v_t, src_v, dst_v, u_v, v_v, o_v,
                         sh_uv):
        cid = lax.axis_index("c")
        tid = lax.axis_index("s")
        wid = tid * NC + cid
        col = tid * n_sl
        pltpu.sync_copy(part_hbm.at[:, pl.ds(col, n_sl)], part_v)
        pltpu.sync_copy(gt_hbm.at[pl.ds(2, 2), pl.ds(col, n_sl)], g_v)

        @plsc.parallel_loop(0, n_sl // L, unroll=2)
        def _(i):
            sl = pl.ds(i * L, L)
            den = jnp.maximum(part_v[2, sl] + part_v[5, sl], 1.0)
            u_t[sl] = (part_v[0, sl] + part_v[3, sl]) / den + g_v[0, sl]
            v_t[sl] = (part_v[1, sl] + part_v[4, sl]) / den + g_v[1, sl]

        pltpu.sync_copy(u_t, sh_uv.at[0, pl.ds(col, n_sl)])
        pltpu.sync_copy(v_t, sh_uv.at[1, pl.ds(col, n_sl)])
        plsc.subcore_barrier()

        pltpu.sync_copy(sh_uv.at[0], u_v)
        pltpu.sync_copy(sh_uv.at[1], v_v)
        base = wid * e_per_w
        pltpu.sync_copy(ei_hbm.at[pl.ds(0, 1), pl.ds(base, e_per_w)], src_v)
        pltpu.sync_copy(ei_hbm.at[pl.ds(1, 1), pl.ds(base, e_per_w)], dst_v)

        @plsc.parallel_loop(0, e_per_w // L, unroll=8)
        def _(i):
            sl = pl.ds(i * L, L)
            gu = plsc.load_gather(u_v, [src_v[0, sl]])
            gv = plsc.load_gather(v_v, [dst_v[0, sl]])
            o_v[sl] = gu + gv

        pltpu.sync_copy(o_v, out_hbm.at[pl.ds(base, e_per_w)])

    return uv_gather_kernel


def kernel(x, edge_index, Wl, Wr, b_sage, Wf1, bf1, Wout, bout):
    n, d = x.shape
    n_edges = edge_index.shape[1]
    n_pad = ((n + NW * L - 1) // (NW * L)) * (NW * L)
    e_per_w = n_edges // NW
    assert e_per_w * NW == n_edges and e_per_w % L == 0

    gt = _make_tc_scalars(n_pad)(
        x, Wl, Wr, Wf1, Wout,
        b_sage.reshape(1, -1), bf1.reshape(1, -1), bout.reshape(1, 1))

    part = _make_scatter(e_per_w, n_pad)(edge_index, gt)
    out = _make_uv_gather(e_per_w, n_pad, n_edges)(part, gt, edge_index)
    return out.reshape(n_edges, 1)
